# Initial kernel scaffold; baseline (speedup 1.0000x reference)
import jax
import jax.numpy as jnp
from jax.experimental import pallas as pl


def kernel(inputs):
    raise NotImplementedError("write your pallas kernel here")


# speedup vs baseline: 4.2748x; 4.2748x over previous
import functools

import jax
import jax.numpy as jnp
from jax import lax
from jax.experimental import pallas as pl
from jax.experimental.pallas import tpu as pltpu
from jax.experimental.pallas import tpu_sc as plsc

_PATHS = 64
_T = 32768
_D = 768
_NC = 2
_NS = 16
_NW = _NC * _NS
_ROWS_W = _T // _NW
_CH = 64
_NCHUNK = _ROWS_W // _CH
_NBUF = 2


def _dispatch_body(x_hbm, out_hbm, buf0, buf1, si0, si1, so0, so1):
    wid = lax.axis_index("s") * _NC + lax.axis_index("c")
    base = wid * _ROWS_W
    bufs = (buf0, buf1)
    sem_in = (si0, si1)
    sem_out = (so0, so1)

    def start_in(i):
        b = i % _NBUF
        cp = pltpu.make_async_copy(
            x_hbm.at[pl.ds(base + i * _CH, _CH)], bufs[b], sem_in[b])
        cp.start()
        return cp

    def start_out(i):
        b = i % _NBUF
        cp = pltpu.make_async_copy(
            bufs[b], out_hbm.at[pl.ds(base + i * _CH, _CH)], sem_out[b])
        cp.start()
        return cp

    in_cp = [None] * _NCHUNK
    out_cp = [None] * _NCHUNK
    in_cp[0] = start_in(0)
    for i in range(_NCHUNK):
        nxt = i + 1
        if nxt < _NCHUNK:
            if nxt >= _NBUF:
                out_cp[nxt - _NBUF].wait()
            in_cp[nxt] = start_in(nxt)
        in_cp[i].wait()
        out_cp[i] = start_out(i)
    for j in range(max(0, _NCHUNK - _NBUF), _NCHUNK):
        out_cp[j].wait()


@jax.jit
def kernel(inputs):
    mesh = plsc.VectorSubcoreMesh(
        core_axis_name="c", subcore_axis_name="s",
        num_cores=_NC, num_subcores=_NS)
    routed_flat = pl.kernel(
        _dispatch_body,
        out_type=jax.ShapeDtypeStruct((_T, _D), jnp.float32),
        mesh=mesh,
        scratch_types=[
            pltpu.VMEM((_CH, _D), jnp.float32),
            pltpu.VMEM((_CH, _D), jnp.float32),
            pltpu.SemaphoreType.DMA,
            pltpu.SemaphoreType.DMA,
            pltpu.SemaphoreType.DMA,
            pltpu.SemaphoreType.DMA,
        ],
    )(inputs)
    return routed_flat.reshape(_PATHS, _T // _PATHS, _D)
